# Initial kernel scaffold; baseline (speedup 1.0000x reference)
import jax
import jax.numpy as jnp
from jax.experimental import pallas as pl


def kernel(tm, spec, cwt, scat, snr, params):
    raise NotImplementedError("write your pallas kernel here")


# speedup vs baseline: 2.9863x; 2.9863x over previous
import jax
import numpy as np
import jax.numpy as jnp
from jax import lax
from jax.experimental import pallas as pl
from jax.experimental.pallas import tpu as pltpu

B, S, L, T = 32, 16, 128, 1088
FEAT = 128
HEADS_TF = 4
DH = FEAT // HEADS_TF
GNN_H, GNN_HEADS = 128, 4
NUM_CLASSES = 24
SCAT_CH = 7
STEP = 64
NSEG = 8
NBLK = (B * S) // NSEG

f32 = jnp.float32
bf16 = jnp.bfloat16


def _b(x):
    return x.astype(bf16)


def _dot(a, b):
    return jnp.dot(a, b, preferred_element_type=f32)


def _dotf(a, b):
    return jnp.dot(a, b, precision=lax.Precision.HIGHEST,
                   preferred_element_type=f32)


def _ae_kernel(x_ref, w1, b1, w2, b2, w3, b3, w4, b4, out_ref):
    x = x_ref[0]

    def taps3(h):
        z = jnp.zeros((h.shape[0], 1), h.dtype)
        hm = jnp.concatenate([z, h[:, :-1]], axis=1)
        hp = jnp.concatenate([h[:, 1:], z], axis=1)
        return jnp.concatenate([hm, h, hp], axis=0)

    def conv(h, w, b):
        return _dot(w[...], taps3(_b(h))) + b[...]

    h = jax.nn.relu(conv(x, w1, b1))
    h = jax.nn.relu(conv(h, w2, b2))
    r = jax.nn.relu(conv(h, w3, b3))
    recon = conv(r, w4, b4)
    out_ref[0] = x - recon


def _enc_kernel(xsh_ref, scat_ref, wmsc, bmsc, bng, bnb, bnrm, bnrv,
                ln1g, ln1b, wqkv, bqkv, wo, bo, ln2g, ln2b,
                mw1, mb1, mw2, mb2, scw, scb, out_ref):
    xsh = xsh_ref[...].reshape(NSEG * L, 7 * 4)
    y = _dot(xsh, wmsc[0]) + bmsc[0]
    y = (y - bnrm[0]) / jnp.sqrt(bnrv[0] + 1e-5) * bng[0] + bnb[0]
    y = jax.nn.relu(y)

    def layernorm(h, g, b):
        m = jnp.mean(h, axis=-1, keepdims=True)
        v = jnp.mean((h - m) ** 2, axis=-1, keepdims=True)
        return (h - m) / jnp.sqrt(v + 1e-5) * g + b

    h = layernorm(y, ln1g[0], ln1b[0])
    qkv = (_dot(_b(h), wqkv[0]) + bqkv[0]).reshape(NSEG, L, 3 * FEAT)
    outs = []
    for hd in range(HEADS_TF):
        q = _b(qkv[:, :, hd * DH:(hd + 1) * DH])
        k = _b(qkv[:, :, FEAT + hd * DH:FEAT + (hd + 1) * DH])
        v = _b(qkv[:, :, 2 * FEAT + hd * DH:2 * FEAT + (hd + 1) * DH])
        s = lax.dot_general(q, k, (((2,), (2,)), ((0,), (0,))),
                            preferred_element_type=f32)
        s = s / jnp.sqrt(jnp.float32(DH))
        s = s - jnp.max(s, axis=-1, keepdims=True)
        e = jnp.exp(s)
        a = e / jnp.sum(e, axis=-1, keepdims=True)
        outs.append(lax.dot_general(_b(a), v, (((2,), (1,)), ((0,), (0,))),
                                    preferred_element_type=f32))
    o = jnp.concatenate(outs, axis=-1).reshape(NSEG * L, FEAT)
    x1 = y + (_dot(_b(o), wo[0]) + bo[0])
    h2 = layernorm(x1, ln2g[0], ln2b[0])
    g = _dot(_b(h2), mw1[0]) + mb1[0]
    g = 0.5 * g * (1.0 + lax.erf(g / jnp.sqrt(jnp.float32(2.0))))
    x2 = x1 + (_dot(_b(g), mw2[0]) + mb2[0])
    tm_feat = jnp.mean(x2.reshape(NSEG, L, FEAT), axis=1)
    scat_feat = jax.nn.relu(_dot(scat_ref[...], scw[0]) + scb[0])
    out_ref[0] = tm_feat + scat_feat


def _spec_kernel(xsp_ref, mask_ref, w1, b1, w2, b2, out_ref):
    y1 = jax.nn.relu(_dot(w1[0], xsp_ref[...]) + b1[0])
    s = _dot(_b(y1).reshape(16 * B, 1024), mask_ref[...])
    s3 = s.reshape(16, B, 9)
    acc = jnp.zeros((B, FEAT), f32)
    for tap in range(9):
        acc = acc + _dotf(s3[:, :, tap].T, w2[0, tap].astype(f32))
    out_ref[0] = acc * (1.0 / 1024.0) + b2[0]


def _leaky(x):
    return jnp.where(x >= 0, x, 0.2 * x)


def _gat(x, wl, bl, wr, att, oh_d, src_mask, heads, out_ch):
    xb = _b(x)
    xl = _dot(xb, wl) + bl
    xsel = jnp.sum(jnp.where(oh_d > 0, x, 0.0), axis=0, keepdims=True)
    xrd = _dot(_b(xsel), wr)
    rows = []
    for hd in range(heads):
        xlh = xl[:, hd * out_ch:(hd + 1) * out_ch]
        t = _leaky(xlh + xrd[:, hd * out_ch:(hd + 1) * out_ch])
        e = _dotf(t, att[hd:hd + 1].T.astype(f32))
        e = jnp.where(src_mask, e, -jnp.inf)
        e = e - jnp.max(e, axis=0, keepdims=True)
        ex = jnp.exp(e)
        alpha = ex / (jnp.sum(ex, axis=0, keepdims=True) + 1e-16)
        rows.append(_dotf(alpha.T, xlh))
    return rows


def _graphnorm(x, w, b, ms):
    mean = jnp.mean(x, axis=0, keepdims=True)
    out = x - mean * ms
    var = jnp.mean(out ** 2, axis=0, keepdims=True)
    return out / jnp.sqrt(var + 1e-5) * w + b


def _elu(x):
    return jnp.where(x > 0, x, jnp.exp(x) - 1.0)


def _head_kernel(feat_ref, spf_ref, snr_ref,
                 gw1, gb1, gw2, gb2, snrw, snrb, pos,
                 g0wl, g0bl, g0wr, g0att, g0bias, gn0w, gn0b, gn0ms,
                 g1wl, g1bl, g1wr, g1att, g1bias, gn1w, gn1b, gn1ms,
                 poolw, poolb, hw1, hb1, hw2, hb2, out_ref):
    f = feat_ref[:, 0] + spf_ref[:, 0]
    snr = snr_ref[0]
    summary = jnp.mean(f[0], axis=0, keepdims=True)
    w_in = _b(jnp.concatenate([summary, snr], axis=1))
    g = jax.nn.relu(_dot(w_in, gw1[...]) + gb1[0])
    glog = _dot(_b(g), gw2[...]) + gb2[0]
    glog = glog - jnp.max(glog, axis=1, keepdims=True)
    gexp = jnp.exp(glog)
    wgt = gexp / jnp.sum(gexp, axis=1, keepdims=True)
    feats = jnp.sum(f * wgt.reshape(3, 1, 1), axis=0)
    snr_e = _b(snr).astype(f32) * _b(snrw[...]).astype(f32) + snrb[...]
    x = (feats + snr_e) + pos[...]

    bprog = pl.program_id(0)
    d = bprog // 2
    parity = bprog % 2
    ridx = lax.broadcasted_iota(jnp.int32, (S, 1), 0)
    oh_d = (ridx == d).astype(f32)
    src_mask = (ridx // 8) == parity

    rows = _gat(x, g0wl[...], g0bl[...], g0wr[...], g0att[...], oh_d,
                src_mask, GNN_HEADS, GNN_H)
    rowcat = jnp.concatenate(rows, axis=-1)
    x = jnp.where(oh_d > 0, jnp.broadcast_to(rowcat, (S, rowcat.shape[1])),
                  0.0) + g0bias[...]
    x = _elu(_graphnorm(x, gn0w[...], gn0b[...], gn0ms[...]))

    rows = _gat(x, g1wl[...], g1bl[...], g1wr[...], g1att[...], oh_d,
                src_mask, GNN_HEADS, GNN_H)
    rowmean = (rows[0] + rows[1] + rows[2] + rows[3]) * 0.25
    x = jnp.where(oh_d > 0, jnp.broadcast_to(rowmean, (S, GNN_H)),
                  0.0) + g1bias[...]
    x = _elu(_graphnorm(x, gn1w[...], gn1b[...], gn1ms[...]))

    score = (_dot(_b(x), poolw[...]) + poolb[...]).reshape(1, S)
    mask = jnp.ones((1, S), f32)
    pooled = jnp.zeros((1, GNN_H), f32)
    for _ in range(S // 2):
        masked = jnp.where(mask > 0, score, -jnp.inf)
        mx = jnp.max(masked, axis=1, keepdims=True)
        is_max = jnp.logical_and(masked == mx, mask > 0).astype(f32)
        tri = (lax.broadcasted_iota(jnp.int32, (S, S), 0)
               <= lax.broadcasted_iota(jnp.int32, (S, S), 1)).astype(f32)
        oh = jnp.logical_and(is_max > 0, _dotf(is_max, tri) == 1.0)
        ohf = oh.astype(f32)
        sel = jnp.sum(jnp.where(ohf.T > 0, x, 0.0), axis=0, keepdims=True)
        pooled = pooled + sel * jnp.tanh(mx)
        mask = mask - ohf
    pooled = pooled * (1.0 / (S // 2))
    h = jax.nn.relu(_dot(_b(pooled), hw1[...]) + hb1[0])
    out_ref[0] = _dot(_b(h), hw2[...]) + hb2[0]


def _conv_w(w):
    return w.transpose(2, 1, 0).reshape(-1, w.shape[0])


def kernel(tm, spec, cwt, scat, snr, params):
    p = params

    ae = p['ae']
    ae_args = [_b(_conv_w(ae['ew1']).T), ae['eb1'][:, None],
               _b(_conv_w(ae['ew2']).T), ae['eb2'][:, None],
               _b(_conv_w(ae['dw1']).T), ae['db1'][:, None],
               _b(_conv_w(ae['dw2']).T), ae['db2'][:, None]]
    z = tm.transpose(0, 2, 1)
    tm_res = pl.pallas_call(
        _ae_kernel,
        grid=(B,),
        out_shape=jax.ShapeDtypeStruct((B, 4, T), f32),
        in_specs=[pl.BlockSpec((1, 4, T), lambda b: (b, 0, 0))]
                 + [pl.BlockSpec(a.shape, lambda b, _n=a.ndim: (0,) * _n)
                    for a in ae_args],
        out_specs=pl.BlockSpec((1, 4, T), lambda b: (b, 0, 0)),
    )(z, *ae_args)

    r = tm_res.transpose(0, 2, 1).reshape(B, T // STEP, STEP, 4)
    win = jnp.concatenate([r[:, :S], r[:, 1:S + 1]], axis=2)
    wf = _b(win.reshape(B * S, L, 4))
    wp = jnp.pad(wf, ((0, 0), (3, 3), (0, 0)))
    xsh = jnp.concatenate([wp[:, 3 + d:3 + d + L, :] for d in range(-3, 4)],
                          axis=-1)

    chs = [FEAT // 3, FEAT // 3, FEAT - 2 * (FEAT // 3)]
    ks = [3, 5, 7]

    def msc_big(d):
        wbig = jnp.zeros((28, FEAT), f32)
        bbig = jnp.concatenate([d['msc_b%d' % j] for j in range(3)])
        coff = 0
        for j in range(3):
            w = d['msc_w%d' % j]
            pad = ks[j] // 2
            blk = jnp.zeros((7, 4, chs[j]), f32)
            blk = blk.at[3 - pad:3 - pad + ks[j]].set(w.transpose(2, 1, 0))
            wbig = wbig.at[:, coff:coff + chs[j]].set(blk.reshape(28, chs[j]))
            coff += chs[j]
        return wbig, bbig[None]

    experts = [p['psk'], p['qam'], p['fsk']]

    def st(fn):
        return jnp.stack([fn(d) for d in experts])

    enc_args = [
        st(lambda d: _b(msc_big(d)[0])), st(lambda d: msc_big(d)[1]),
        st(lambda d: d['bn_g'][None]), st(lambda d: d['bn_b'][None]),
        st(lambda d: d['bn_rm'][None]), st(lambda d: d['bn_rv'][None]),
        st(lambda d: d['ln1_g'][None]), st(lambda d: d['ln1_b'][None]),
        st(lambda d: _b(d['Wqkv'].T)), st(lambda d: d['bqkv'][None]),
        st(lambda d: _b(d['Wo'].T)), st(lambda d: d['bo'][None]),
        st(lambda d: d['ln2_g'][None]), st(lambda d: d['ln2_b'][None]),
        st(lambda d: _b(d['mW1'].T)), st(lambda d: d['mb1'][None]),
        st(lambda d: _b(d['mW2'].T)), st(lambda d: d['mb2'][None]),
        st(lambda d: _b(d['scat_W'].T)), st(lambda d: d['scat_b'][None]),
    ]

    def espec(a):
        shp = (1,) + a.shape[1:]
        return pl.BlockSpec(shp, lambda e, t, _n=len(shp): (e,) + (0,) * (_n - 1))

    scat_flat = _b(scat.reshape(B * S, SCAT_CH))
    feat = pl.pallas_call(
        _enc_kernel,
        grid=(3, NBLK),
        out_shape=jax.ShapeDtypeStruct((3, B * S, FEAT), f32),
        in_specs=[
            pl.BlockSpec((NSEG, L, 28), lambda e, t: (t, 0, 0)),
            pl.BlockSpec((NSEG, SCAT_CH), lambda e, t: (t, 0)),
        ] + [espec(a) for a in enc_args],
        out_specs=pl.BlockSpec((1, NSEG, FEAT), lambda e, t: (e, t, 0)),
    )(xsh, scat_flat, *enc_args)

    spad = jnp.pad(spec, ((0, 0), (1, 1), (1, 1), (0, 0)))
    xsp = _b(jnp.stack([
        spad[:, dh:dh + 32, dw:dw + 32, i].reshape(B * 1024)
        for dh in range(3) for dw in range(3) for i in range(2)
    ]))
    mrows = []
    for dh in range(3):
        for dw in range(3):
            r0, r1 = max(0, dh - 1), min(32, dh + 31)
            c0, c1 = max(0, dw - 1), min(32, dw + 31)
            m = np.zeros((32, 32), np.float32)
            m[r0:r1, c0:c1] = 1.0
            mrows.append(m.reshape(1024))
    mask = _b(jnp.asarray(np.stack(mrows, axis=1)))

    spec_args = [
        st(lambda d: _b(d['cw1'].transpose(0, 2, 3, 1).reshape(16, 18))),
        st(lambda d: d['cb1'][:, None]),
        st(lambda d: _b(d['cw2'].transpose(2, 3, 1, 0).reshape(9, 16, FEAT))),
        st(lambda d: d['cb2'][None]),
    ]

    def sspec(a):
        shp = (1,) + a.shape[1:]
        return pl.BlockSpec(shp, lambda e, _n=len(shp): (e,) + (0,) * (_n - 1))

    spfeat = pl.pallas_call(
        _spec_kernel,
        grid=(3,),
        out_shape=jax.ShapeDtypeStruct((3, B, FEAT), f32),
        in_specs=[pl.BlockSpec((18, B * 1024), lambda e: (0, 0)),
                  pl.BlockSpec((1024, 9), lambda e: (0, 0))]
                 + [sspec(a) for a in spec_args],
        out_specs=pl.BlockSpec((1, B, FEAT), lambda e: (e, 0, 0)),
    )(xsp, mask, *spec_args)

    featb = feat.reshape(3, B, S, FEAT)
    head_args = [
        _b(p['gate_W1'].T), p['gate_b1'][None],
        _b(p['gate_W2'].T), p['gate_b2'][None],
        p['snr_W'].T, p['snr_b'][None], p['pos'],
        _b(p['g0_Wl'].T), p['g0_bl'][None], _b(p['g0_Wr'].T), p['g0_att'],
        p['g0_bias'][None], p['gn0_w'][None], p['gn0_b'][None], p['gn0_ms'][None],
        _b(p['g1_Wl'].T), p['g1_bl'][None], _b(p['g1_Wr'].T), p['g1_att'],
        p['g1_bias'][None], p['gn1_w'][None], p['gn1_b'][None], p['gn1_ms'][None],
        _b(p['pool_W'].T), p['pool_b'][None],
        _b(p['head_W1'].T), p['head_b1'][None],
        _b(p['head_W2'].T), p['head_b2'][None],
    ]

    def hspec(a):
        nd = a.ndim
        return pl.BlockSpec(a.shape, lambda b, _n=nd: (0,) * _n)

    logits = pl.pallas_call(
        _head_kernel,
        grid=(B,),
        out_shape=jax.ShapeDtypeStruct((B, 1, NUM_CLASSES), f32),
        in_specs=[
            pl.BlockSpec((3, 1, S, FEAT), lambda b: (0, b, 0, 0)),
            pl.BlockSpec((3, 1, 1, FEAT), lambda b: (0, b, 0, 0)),
            pl.BlockSpec((1, 1, 1), lambda b: (b, 0, 0)),
        ] + [hspec(a) for a in head_args],
        out_specs=pl.BlockSpec((1, 1, NUM_CLASSES), lambda b: (b, 0, 0)),
    )(featb, spfeat.reshape(3, B, 1, FEAT), snr[:, None, None], *head_args)

    return logits.reshape(B, NUM_CLASSES)
